# baseline (speedup 1.0000x reference)
import functools

import jax
import jax.numpy as jnp
from jax import lax
from jax.experimental import pallas as pl
from jax.experimental.pallas import tpu as pltpu
from jax.experimental.pallas import tpu_sc as plsc

N = 10000
E = 320000
F = 128
NC, NS = 2, 16
NW = NC * NS
EB = 128
NG = 80
IC = 8
EROWS = NW * NG
EPAD = EROWS * EB
NAE = 10240
NBUF = 4
ZCH = 128
NZK = NAE // (ZCH * NS)

_mesh = plsc.VectorSubcoreMesh(core_axis_name="c", subcore_axis_name="s")


@functools.partial(
    pl.kernel,
    out_type=jax.ShapeDtypeStruct((NC, NAE, F), jnp.float32),
    mesh=_mesh,
    scratch_types=[
        pltpu.VMEM((EB,), jnp.int32),
        pltpu.VMEM((EB,), jnp.int32),
        pltpu.VMEM((EB, F), jnp.float32),
        pltpu.VMEM_SHARED((NAE, F), jnp.float32),
        [pltpu.SemaphoreType.DMA] * NBUF,
    ],
)
def _sc_agg(x_hbm, src_hbm, dst_hbm, z_hbm, out_hbm,
            sidx, didx, rows, acc, sems):
    cid = lax.axis_index("c")
    sid = lax.axis_index("s")
    wid = cid * NS + sid
    rbase = wid * NG

    for k in range(NZK):
        if k >= NBUF:
            pltpu.make_async_copy(z_hbm.at[pl.ds(0, ZCH)],
                                  acc.at[pl.ds(0, ZCH)], sems[k % NBUF]).wait()
        row = (k * NS + sid) * ZCH
        pltpu.async_copy(z_hbm.at[pl.ds(row, ZCH)],
                         acc.at[pl.ds(row, ZCH)], sems[k % NBUF])
    for b in range(NBUF):
        pltpu.make_async_copy(z_hbm.at[pl.ds(0, ZCH)],
                              acc.at[pl.ds(0, ZCH)], sems[b]).wait()
    plsc.subcore_barrier()

    ebase = wid * (NG * EB)

    def _batch(g, carry):
        off = ebase + g * EB
        pltpu.sync_copy(src_hbm.at[pl.ds(off, EB)], sidx)
        pltpu.sync_copy(dst_hbm.at[pl.ds(off, EB)], didx)
        pltpu.async_copy(x_hbm.at[sidx], rows, sems[0]).wait()
        pltpu.async_copy(rows, acc.at[didx], sems[0], add=True).wait()
        return carry

    lax.fori_loop(0, NG, _batch, 0)
    plsc.subcore_barrier()

    for k in range(NZK):
        if k >= NBUF:
            pltpu.make_async_copy(acc.at[pl.ds(0, ZCH)],
                                  out_hbm.at[cid, pl.ds(0, ZCH)], sems[k % NBUF]).wait()
        row = (k * NS + sid) * ZCH
        pltpu.async_copy(acc.at[pl.ds(row, ZCH)],
                         out_hbm.at[cid, pl.ds(row, ZCH)], sems[k % NBUF])
    for b in range(NBUF):
        pltpu.make_async_copy(acc.at[pl.ds(0, ZCH)],
                              out_hbm.at[cid, pl.ds(0, ZCH)], sems[b]).wait()


def _tc_layer_call(y, x, W, Ws, b, *, final):
    M, Fin = x.shape
    Fo = W.shape[1]
    BM = 1000

    def body(ya_ref, yb_ref, x_ref, W_ref, Ws_ref, b_ref, o_ref):
        ys = ya_ref[0] + yb_ref[0]
        acc = jnp.dot(ys, W_ref[...], preferred_element_type=jnp.float32)
        acc += jnp.dot(x_ref[...], Ws_ref[...], preferred_element_type=jnp.float32)
        logits = acc + b_ref[...]
        if final:
            m = jnp.max(logits, axis=-1, keepdims=True)
            z = logits - m
            lse = jnp.log(jnp.sum(jnp.exp(z), axis=-1, keepdims=True))
            o_ref[...] = z - lse
        else:
            o_ref[...] = logits

    return pl.pallas_call(
        body,
        grid=(M // BM,),
        in_specs=[
            pl.BlockSpec((1, BM, Fin), lambda i: (0, i, 0)),
            pl.BlockSpec((1, BM, Fin), lambda i: (1, i, 0)),
            pl.BlockSpec((BM, Fin), lambda i: (i, 0)),
            pl.BlockSpec((Fin, Fo), lambda i: (0, 0)),
            pl.BlockSpec((Fin, Fo), lambda i: (0, 0)),
            pl.BlockSpec((1, Fo), lambda i: (0, 0)),
        ],
        out_specs=pl.BlockSpec((BM, Fo), lambda i: (i, 0)),
        out_shape=jax.ShapeDtypeStruct((M, Fo), jnp.float32),
    )(y, y, x, W, Ws, b.reshape(1, Fo))


def kernel(fea, edge_index, W_in, Ws_in, b_in, W_mid, Ws_mid, b_mid,
           W_out, Ws_out, b_out):
    pad = EPAD - E
    src = jnp.concatenate([edge_index[0], jnp.zeros((pad,), jnp.int32)])
    trash = N + jax.lax.rem(jnp.arange(pad, dtype=jnp.int32), NAE - N)
    dst = jnp.concatenate([edge_index[1], trash])

    z = jnp.zeros((NAE, F), jnp.float32)
    y = _sc_agg(fea, src, dst, z)
    x1 = _tc_layer_call(y, fea, W_in, Ws_in, b_in, final=False)
    y = _sc_agg(x1, src, dst, z)
    x2 = _tc_layer_call(y, x1, W_mid, Ws_mid, b_mid, final=False)
    y = _sc_agg(x2, src, dst, z)
    return _tc_layer_call(y, x2, W_out, Ws_out, b_out, final=True)


# speedup vs baseline: 2.1634x; 2.1634x over previous
import functools

import jax
import jax.numpy as jnp
from jax import lax
from jax.experimental import pallas as pl
from jax.experimental.pallas import tpu as pltpu
from jax.experimental.pallas import tpu_sc as plsc

N = 10000
E = 320000
F = 128
NC, NS = 2, 16
NW = NC * NS
EB = 128
NG = 80
IC = 8
EROWS = NW * NG
EPAD = EROWS * EB
NAE = 10240
NBUF = 4
ZCH = 128
NZK = NAE // (ZCH * NS)

_mesh = plsc.VectorSubcoreMesh(core_axis_name="c", subcore_axis_name="s")


@functools.partial(
    pl.kernel,
    out_type=jax.ShapeDtypeStruct((NC, NAE, F), jnp.float32),
    mesh=_mesh,
    scratch_types=[
        pltpu.VMEM((EB,), jnp.int32),
        pltpu.VMEM((EB,), jnp.int32),
        pltpu.VMEM((EB, F), jnp.float32),
        pltpu.VMEM_SHARED((NAE, F), jnp.float32),
        [pltpu.SemaphoreType.DMA] * NBUF,
    ],
)
def _sc_agg(x_hbm, src_hbm, dst_hbm, z_hbm, out_hbm,
            sidx, didx, rows, acc, sems):
    cid = lax.axis_index("c")
    sid = lax.axis_index("s")
    wid = cid * NS + sid
    rbase = wid * NG

    for k in range(NZK):
        if k >= NBUF:
            pltpu.make_async_copy(z_hbm.at[pl.ds(0, ZCH)],
                                  acc.at[pl.ds(0, ZCH)], sems[k % NBUF]).wait()
        row = (k * NS + sid) * ZCH
        pltpu.async_copy(z_hbm.at[pl.ds(row, ZCH)],
                         acc.at[pl.ds(row, ZCH)], sems[k % NBUF])
    for b in range(NBUF):
        pltpu.make_async_copy(z_hbm.at[pl.ds(0, ZCH)],
                              acc.at[pl.ds(0, ZCH)], sems[b]).wait()
    plsc.subcore_barrier()

    ebase = wid * (NG * EB)

    def _batch(g, carry):
        off = ebase + g * EB
        pltpu.sync_copy(src_hbm.at[pl.ds(off, EB)], sidx)
        pltpu.sync_copy(dst_hbm.at[pl.ds(off, EB)], didx)
        pltpu.async_copy(x_hbm.at[sidx], rows, sems[0]).wait()
        pltpu.async_copy(rows, acc.at[didx], sems[0], add=True).wait()
        return carry

    lax.fori_loop(0, NG, _batch, 0)
    plsc.subcore_barrier()

    for k in range(NZK):
        if k >= NBUF:
            pltpu.make_async_copy(acc.at[pl.ds(0, ZCH)],
                                  out_hbm.at[cid, pl.ds(0, ZCH)], sems[k % NBUF]).wait()
        row = (k * NS + sid) * ZCH
        pltpu.async_copy(acc.at[pl.ds(row, ZCH)],
                         out_hbm.at[cid, pl.ds(row, ZCH)], sems[k % NBUF])
    for b in range(NBUF):
        pltpu.make_async_copy(acc.at[pl.ds(0, ZCH)],
                              out_hbm.at[cid, pl.ds(0, ZCH)], sems[b]).wait()


def _tc_layer_call(y, x, W, Ws, b, *, final):
    M, Fin = x.shape
    Fo = W.shape[1]
    BM = 1000

    def body(ya_ref, yb_ref, x_ref, W_ref, Ws_ref, b_ref, o_ref):
        ys = ya_ref[0] + yb_ref[0]
        acc = jnp.dot(ys, W_ref[...], preferred_element_type=jnp.float32)
        acc += jnp.dot(x_ref[...], Ws_ref[...], preferred_element_type=jnp.float32)
        logits = acc + b_ref[...]
        if final:
            m = jnp.max(logits, axis=-1, keepdims=True)
            z = logits - m
            lse = jnp.log(jnp.sum(jnp.exp(z), axis=-1, keepdims=True))
            o_ref[...] = z - lse
        else:
            o_ref[...] = logits

    return pl.pallas_call(
        body,
        grid=(M // BM,),
        in_specs=[
            pl.BlockSpec((1, BM, Fin), lambda i: (0, i, 0)),
            pl.BlockSpec((1, BM, Fin), lambda i: (1, i, 0)),
            pl.BlockSpec((BM, Fin), lambda i: (i, 0)),
            pl.BlockSpec((Fin, Fo), lambda i: (0, 0)),
            pl.BlockSpec((Fin, Fo), lambda i: (0, 0)),
            pl.BlockSpec((1, Fo), lambda i: (0, 0)),
        ],
        out_specs=pl.BlockSpec((BM, Fo), lambda i: (i, 0)),
        out_shape=jax.ShapeDtypeStruct((M, Fo), jnp.float32),
    )(y, y, x, W, Ws, b.reshape(1, Fo))


def kernel(fea, edge_index, W_in, Ws_in, b_in, W_mid, Ws_mid, b_mid,
           W_out, Ws_out, b_out):
    pad = EPAD - E
    ar = jnp.arange(pad, dtype=jnp.int32)
    src = jnp.concatenate([edge_index[0], jax.lax.rem(ar * 41, N)])
    dst = jnp.concatenate([edge_index[1], N + jax.lax.rem(ar, NAE - N)])

    z = jnp.zeros((NAE, F), jnp.float32)
    y = _sc_agg(fea, src, dst, z)
    x1 = _tc_layer_call(y, fea, W_in, Ws_in, b_in, final=False)
    y = _sc_agg(x1, src, dst, z)
    x2 = _tc_layer_call(y, x1, W_mid, Ws_mid, b_mid, final=False)
    y = _sc_agg(x2, src, dst, z)
    return _tc_layer_call(y, x2, W_out, Ws_out, b_out, final=True)


# speedup vs baseline: 2.7915x; 1.2903x over previous
import functools

import jax
import jax.numpy as jnp
from jax import lax
from jax.experimental import pallas as pl
from jax.experimental.pallas import tpu as pltpu
from jax.experimental.pallas import tpu_sc as plsc

N = 10000
E = 320000
F = 128
NC, NS = 2, 16
NW = NC * NS
EB = 256
NG = 40
IC = 8
EROWS = NW * NG
EPAD = EROWS * EB
NAE = 10240
NBUF = 4
ZCH = 128
NZK = NAE // (ZCH * NS)

_mesh = plsc.VectorSubcoreMesh(core_axis_name="c", subcore_axis_name="s")


@functools.partial(
    pl.kernel,
    out_type=jax.ShapeDtypeStruct((NC, NAE, F), jnp.float32),
    mesh=_mesh,
    scratch_types=[
        pltpu.VMEM((EB,), jnp.int32),
        pltpu.VMEM((EB,), jnp.int32),
        pltpu.VMEM((EB, F), jnp.float32),
        pltpu.VMEM_SHARED((NAE, F), jnp.float32),
        [pltpu.SemaphoreType.DMA] * NBUF,
    ],
)
def _sc_agg(x_hbm, src_hbm, dst_hbm, z_hbm, out_hbm,
            sidx, didx, rows, acc, sems):
    cid = lax.axis_index("c")
    sid = lax.axis_index("s")
    wid = cid * NS + sid
    rbase = wid * NG

    for k in range(NZK):
        if k >= NBUF:
            pltpu.make_async_copy(z_hbm.at[pl.ds(0, ZCH)],
                                  acc.at[pl.ds(0, ZCH)], sems[k % NBUF]).wait()
        row = (k * NS + sid) * ZCH
        pltpu.async_copy(z_hbm.at[pl.ds(row, ZCH)],
                         acc.at[pl.ds(row, ZCH)], sems[k % NBUF])
    for b in range(NBUF):
        pltpu.make_async_copy(z_hbm.at[pl.ds(0, ZCH)],
                              acc.at[pl.ds(0, ZCH)], sems[b]).wait()
    plsc.subcore_barrier()

    ebase = wid * (NG * EB)

    def _batch(g, carry):
        off = ebase + g * EB
        pltpu.sync_copy(src_hbm.at[pl.ds(off, EB)], sidx)
        pltpu.sync_copy(dst_hbm.at[pl.ds(off, EB)], didx)
        pltpu.async_copy(x_hbm.at[sidx], rows, sems[0]).wait()
        pltpu.async_copy(rows, acc.at[didx], sems[0], add=True).wait()
        return carry

    lax.fori_loop(0, NG, _batch, 0)
    plsc.subcore_barrier()

    for k in range(NZK):
        if k >= NBUF:
            pltpu.make_async_copy(acc.at[pl.ds(0, ZCH)],
                                  out_hbm.at[cid, pl.ds(0, ZCH)], sems[k % NBUF]).wait()
        row = (k * NS + sid) * ZCH
        pltpu.async_copy(acc.at[pl.ds(row, ZCH)],
                         out_hbm.at[cid, pl.ds(row, ZCH)], sems[k % NBUF])
    for b in range(NBUF):
        pltpu.make_async_copy(acc.at[pl.ds(0, ZCH)],
                              out_hbm.at[cid, pl.ds(0, ZCH)], sems[b]).wait()


def _tc_layer_call(y, x, W, Ws, b, *, final):
    M, Fin = x.shape
    Fo = W.shape[1]
    BM = 1000

    def body(ya_ref, yb_ref, x_ref, W_ref, Ws_ref, b_ref, o_ref):
        ys = ya_ref[0] + yb_ref[0]
        acc = jnp.dot(ys, W_ref[...], preferred_element_type=jnp.float32)
        acc += jnp.dot(x_ref[...], Ws_ref[...], preferred_element_type=jnp.float32)
        logits = acc + b_ref[...]
        if final:
            m = jnp.max(logits, axis=-1, keepdims=True)
            z = logits - m
            lse = jnp.log(jnp.sum(jnp.exp(z), axis=-1, keepdims=True))
            o_ref[...] = z - lse
        else:
            o_ref[...] = logits

    return pl.pallas_call(
        body,
        grid=(M // BM,),
        in_specs=[
            pl.BlockSpec((1, BM, Fin), lambda i: (0, i, 0)),
            pl.BlockSpec((1, BM, Fin), lambda i: (1, i, 0)),
            pl.BlockSpec((BM, Fin), lambda i: (i, 0)),
            pl.BlockSpec((Fin, Fo), lambda i: (0, 0)),
            pl.BlockSpec((Fin, Fo), lambda i: (0, 0)),
            pl.BlockSpec((1, Fo), lambda i: (0, 0)),
        ],
        out_specs=pl.BlockSpec((BM, Fo), lambda i: (i, 0)),
        out_shape=jax.ShapeDtypeStruct((M, Fo), jnp.float32),
    )(y, y, x, W, Ws, b.reshape(1, Fo))


def kernel(fea, edge_index, W_in, Ws_in, b_in, W_mid, Ws_mid, b_mid,
           W_out, Ws_out, b_out):
    pad = EPAD - E
    ar = jnp.arange(pad, dtype=jnp.int32)
    src = jnp.concatenate([edge_index[0], jax.lax.rem(ar * 41, N)])
    dst = jnp.concatenate([edge_index[1], N + jax.lax.rem(ar, NAE - N)])

    z = jnp.zeros((NAE, F), jnp.float32)
    y = _sc_agg(fea, src, dst, z)
    x1 = _tc_layer_call(y, fea, W_in, Ws_in, b_in, final=False)
    y = _sc_agg(x1, src, dst, z)
    x2 = _tc_layer_call(y, x1, W_mid, Ws_mid, b_mid, final=False)
    y = _sc_agg(x2, src, dst, z)
    return _tc_layer_call(y, x2, W_out, Ws_out, b_out, final=True)


# speedup vs baseline: 3.4119x; 1.2223x over previous
import functools

import jax
import jax.numpy as jnp
from jax import lax
from jax.experimental import pallas as pl
from jax.experimental.pallas import tpu as pltpu
from jax.experimental.pallas import tpu_sc as plsc

N = 10000
E = 320000
F = 128
NC, NS = 2, 16
NW = NC * NS
EB = 256
NG = 40
IC = 8
EROWS = NW * NG
EPAD = EROWS * EB
NAE = 10240
NBUF = 4
ZCH = 128
NZK = NAE // (ZCH * NS)

_mesh = plsc.VectorSubcoreMesh(core_axis_name="c", subcore_axis_name="s")


@functools.partial(
    pl.kernel,
    out_type=jax.ShapeDtypeStruct((NC, NAE, F), jnp.float32),
    mesh=_mesh,
    scratch_types=[
        [pltpu.VMEM((EB,), jnp.int32)] * 2,
        [pltpu.VMEM((EB,), jnp.int32)] * 2,
        pltpu.VMEM((EB, F), jnp.float32),
        pltpu.VMEM_SHARED((NAE, F), jnp.float32),
        [pltpu.SemaphoreType.DMA] * NBUF,
        pltpu.SemaphoreType.DMA,
    ],
)
def _sc_agg(x_hbm, src_hbm, dst_hbm, z_hbm, out_hbm,
            sidx, didx, rows, acc, sems, sem_i):
    cid = lax.axis_index("c")
    sid = lax.axis_index("s")
    wid = cid * NS + sid
    rbase = wid * NG

    for k in range(NZK):
        if k >= NBUF:
            pltpu.make_async_copy(z_hbm.at[pl.ds(0, ZCH)],
                                  acc.at[pl.ds(0, ZCH)], sems[k % NBUF]).wait()
        row = (k * NS + sid) * ZCH
        pltpu.async_copy(z_hbm.at[pl.ds(row, ZCH)],
                         acc.at[pl.ds(row, ZCH)], sems[k % NBUF])
    for b in range(NBUF):
        pltpu.make_async_copy(z_hbm.at[pl.ds(0, ZCH)],
                              acc.at[pl.ds(0, ZCH)], sems[b]).wait()
    plsc.subcore_barrier()

    ebase = wid * (NG * EB)
    pltpu.sync_copy(src_hbm.at[pl.ds(ebase, EB)], sidx[0])
    pltpu.sync_copy(dst_hbm.at[pl.ds(ebase, EB)], didx[0])

    def _batch2(i, carry):
        for b in range(2):
            g = i * 2 + b
            nxt = 1 - b

            @pl.when(g + 1 < NG)
            def _():
                off = ebase + (g + 1) * EB
                pltpu.async_copy(src_hbm.at[pl.ds(off, EB)], sidx[nxt], sem_i)
                pltpu.async_copy(dst_hbm.at[pl.ds(off, EB)], didx[nxt], sem_i)

            pltpu.async_copy(x_hbm.at[sidx[b]], rows, sems[0]).wait()
            pltpu.async_copy(rows, acc.at[didx[b]], sems[0], add=True).wait()

            @pl.when(g + 1 < NG)
            def _():
                pltpu.make_async_copy(src_hbm.at[pl.ds(0, EB)], sidx[nxt],
                                      sem_i).wait()
                pltpu.make_async_copy(dst_hbm.at[pl.ds(0, EB)], didx[nxt],
                                      sem_i).wait()
        return carry

    lax.fori_loop(0, NG // 2, _batch2, 0)
    plsc.subcore_barrier()

    for k in range(NZK):
        if k >= NBUF:
            pltpu.make_async_copy(acc.at[pl.ds(0, ZCH)],
                                  out_hbm.at[cid, pl.ds(0, ZCH)], sems[k % NBUF]).wait()
        row = (k * NS + sid) * ZCH
        pltpu.async_copy(acc.at[pl.ds(row, ZCH)],
                         out_hbm.at[cid, pl.ds(row, ZCH)], sems[k % NBUF])
    for b in range(NBUF):
        pltpu.make_async_copy(acc.at[pl.ds(0, ZCH)],
                              out_hbm.at[cid, pl.ds(0, ZCH)], sems[b]).wait()


def _tc_layer_call(y, x, W, Ws, b, *, final):
    M, Fin = x.shape
    Fo = W.shape[1]
    BM = 2000

    def body(ya_ref, yb_ref, x_ref, W_ref, Ws_ref, b_ref, o_ref):
        ys = ya_ref[0] + yb_ref[0]
        acc = jnp.dot(ys, W_ref[...], preferred_element_type=jnp.float32)
        acc += jnp.dot(x_ref[...], Ws_ref[...], preferred_element_type=jnp.float32)
        logits = acc + b_ref[...]
        if final:
            m = jnp.max(logits, axis=-1, keepdims=True)
            z = logits - m
            lse = jnp.log(jnp.sum(jnp.exp(z), axis=-1, keepdims=True))
            o_ref[...] = z - lse
        else:
            o_ref[...] = logits

    return pl.pallas_call(
        body,
        grid=(M // BM,),
        in_specs=[
            pl.BlockSpec((1, BM, Fin), lambda i: (0, i, 0)),
            pl.BlockSpec((1, BM, Fin), lambda i: (1, i, 0)),
            pl.BlockSpec((BM, Fin), lambda i: (i, 0)),
            pl.BlockSpec((Fin, Fo), lambda i: (0, 0)),
            pl.BlockSpec((Fin, Fo), lambda i: (0, 0)),
            pl.BlockSpec((1, Fo), lambda i: (0, 0)),
        ],
        out_specs=pl.BlockSpec((BM, Fo), lambda i: (i, 0)),
        out_shape=jax.ShapeDtypeStruct((M, Fo), jnp.float32),
    )(y, y, x, W, Ws, b.reshape(1, Fo))


def kernel(fea, edge_index, W_in, Ws_in, b_in, W_mid, Ws_mid, b_mid,
           W_out, Ws_out, b_out):
    pad = EPAD - E
    ar = jnp.arange(pad, dtype=jnp.int32)
    src = jnp.concatenate([edge_index[0], jax.lax.rem(ar * 41, N)])
    dst = jnp.concatenate([edge_index[1], N + jax.lax.rem(ar, NAE - N)])

    z = jnp.zeros((NAE, F), jnp.float32)
    y = _sc_agg(fea, src, dst, z)
    x1 = _tc_layer_call(y, fea, W_in, Ws_in, b_in, final=False)
    y = _sc_agg(x1, src, dst, z)
    x2 = _tc_layer_call(y, x1, W_mid, Ws_mid, b_mid, final=False)
    y = _sc_agg(x2, src, dst, z)
    return _tc_layer_call(y, x2, W_out, Ws_out, b_out, final=True)


# speedup vs baseline: 4.1169x; 1.2066x over previous
import functools

import jax
import jax.numpy as jnp
from jax import lax
from jax.experimental import pallas as pl
from jax.experimental.pallas import tpu as pltpu
from jax.experimental.pallas import tpu_sc as plsc

N = 10000
E = 320000
F = 128
NC, NS = 2, 16
NW = NC * NS
EB = 160
NG = 64
IC = 8
EROWS = NW * NG
EPAD = EROWS * EB
NAE = 10240
NBUF = 4
ZCH = 128
NZK = NAE // (ZCH * NS)

_mesh = plsc.VectorSubcoreMesh(core_axis_name="c", subcore_axis_name="s")


@functools.partial(
    pl.kernel,
    out_type=jax.ShapeDtypeStruct((NC, NAE, F), jnp.float32),
    mesh=_mesh,
    scratch_types=[
        [pltpu.VMEM((EB,), jnp.int32)] * 2,
        [pltpu.VMEM((EB,), jnp.int32)] * 2,
        [pltpu.VMEM((EB, F), jnp.float32)] * 2,
        pltpu.VMEM_SHARED((NAE, F), jnp.float32),
        [pltpu.SemaphoreType.DMA] * NBUF,
        pltpu.SemaphoreType.DMA,
    ],
)
def _sc_agg(x_hbm, src_hbm, dst_hbm, z_hbm, out_hbm,
            sidx, didx, rows, acc, sems, sem_i):
    cid = lax.axis_index("c")
    sid = lax.axis_index("s")
    wid = cid * NS + sid
    rbase = wid * NG

    for k in range(NZK):
        if k >= NBUF:
            pltpu.make_async_copy(z_hbm.at[pl.ds(0, ZCH)],
                                  acc.at[pl.ds(0, ZCH)], sems[k % NBUF]).wait()
        row = (k * NS + sid) * ZCH
        pltpu.async_copy(z_hbm.at[pl.ds(row, ZCH)],
                         acc.at[pl.ds(row, ZCH)], sems[k % NBUF])
    for b in range(NBUF):
        pltpu.make_async_copy(z_hbm.at[pl.ds(0, ZCH)],
                              acc.at[pl.ds(0, ZCH)], sems[b]).wait()
    plsc.subcore_barrier()

    ebase = wid * (NG * EB)
    pltpu.sync_copy(src_hbm.at[pl.ds(ebase, EB)], sidx[0])
    pltpu.sync_copy(dst_hbm.at[pl.ds(ebase, EB)], didx[0])

    pltpu.async_copy(x_hbm.at[sidx[0]], rows[0], sems[0])

    def _batch2(i, carry):
        for b in range(2):
            g = i * 2 + b
            nxt = 1 - b

            @pl.when(g + 1 < NG)
            def _():
                off = ebase + (g + 1) * EB
                pltpu.async_copy(src_hbm.at[pl.ds(off, EB)], sidx[nxt], sem_i)
                pltpu.async_copy(dst_hbm.at[pl.ds(off, EB)], didx[nxt], sem_i)

            pltpu.make_async_copy(x_hbm.at[sidx[b]], rows[b], sems[0]).wait()
            pltpu.async_copy(rows[b], acc.at[didx[b]], sems[1], add=True)

            @pl.when(g + 1 < NG)
            def _():
                pltpu.make_async_copy(src_hbm.at[pl.ds(0, EB)], sidx[nxt],
                                      sem_i).wait()
                pltpu.make_async_copy(dst_hbm.at[pl.ds(0, EB)], didx[nxt],
                                      sem_i).wait()
                pltpu.async_copy(x_hbm.at[sidx[nxt]], rows[nxt], sems[0])

            pltpu.make_async_copy(rows[b], acc.at[pl.ds(0, EB)], sems[1]).wait()
        return carry

    lax.fori_loop(0, NG // 2, _batch2, 0)
    plsc.subcore_barrier()

    for k in range(NZK):
        if k >= NBUF:
            pltpu.make_async_copy(acc.at[pl.ds(0, ZCH)],
                                  out_hbm.at[cid, pl.ds(0, ZCH)], sems[k % NBUF]).wait()
        row = (k * NS + sid) * ZCH
        pltpu.async_copy(acc.at[pl.ds(row, ZCH)],
                         out_hbm.at[cid, pl.ds(row, ZCH)], sems[k % NBUF])
    for b in range(NBUF):
        pltpu.make_async_copy(acc.at[pl.ds(0, ZCH)],
                              out_hbm.at[cid, pl.ds(0, ZCH)], sems[b]).wait()


def _tc_layer_call(y, x, W, Ws, b, *, final):
    M, Fin = x.shape
    Fo = W.shape[1]
    BM = 2000

    def body(ya_ref, yb_ref, x_ref, W_ref, Ws_ref, b_ref, o_ref):
        ys = ya_ref[0] + yb_ref[0]
        acc = jnp.dot(ys, W_ref[...], preferred_element_type=jnp.float32)
        acc += jnp.dot(x_ref[...], Ws_ref[...], preferred_element_type=jnp.float32)
        logits = acc + b_ref[...]
        if final:
            m = jnp.max(logits, axis=-1, keepdims=True)
            z = logits - m
            lse = jnp.log(jnp.sum(jnp.exp(z), axis=-1, keepdims=True))
            o_ref[...] = z - lse
        else:
            o_ref[...] = logits

    return pl.pallas_call(
        body,
        grid=(M // BM,),
        in_specs=[
            pl.BlockSpec((1, BM, Fin), lambda i: (0, i, 0)),
            pl.BlockSpec((1, BM, Fin), lambda i: (1, i, 0)),
            pl.BlockSpec((BM, Fin), lambda i: (i, 0)),
            pl.BlockSpec((Fin, Fo), lambda i: (0, 0)),
            pl.BlockSpec((Fin, Fo), lambda i: (0, 0)),
            pl.BlockSpec((1, Fo), lambda i: (0, 0)),
        ],
        out_specs=pl.BlockSpec((BM, Fo), lambda i: (i, 0)),
        out_shape=jax.ShapeDtypeStruct((M, Fo), jnp.float32),
    )(y, y, x, W, Ws, b.reshape(1, Fo))


def kernel(fea, edge_index, W_in, Ws_in, b_in, W_mid, Ws_mid, b_mid,
           W_out, Ws_out, b_out):
    pad = EPAD - E
    ar = jnp.arange(pad, dtype=jnp.int32)
    src = jnp.concatenate([edge_index[0], jax.lax.rem(ar * 41, N)])
    dst = jnp.concatenate([edge_index[1], N + jax.lax.rem(ar, NAE - N)])

    z = jnp.zeros((NAE, F), jnp.float32)
    y = _sc_agg(fea, src, dst, z)
    x1 = _tc_layer_call(y, fea, W_in, Ws_in, b_in, final=False)
    y = _sc_agg(x1, src, dst, z)
    x2 = _tc_layer_call(y, x1, W_mid, Ws_mid, b_mid, final=False)
    y = _sc_agg(x2, src, dst, z)
    return _tc_layer_call(y, x2, W_out, Ws_out, b_out, final=True)


# speedup vs baseline: 4.1827x; 1.0160x over previous
import functools

import jax
import jax.numpy as jnp
from jax import lax
from jax.experimental import pallas as pl
from jax.experimental.pallas import tpu as pltpu
from jax.experimental.pallas import tpu_sc as plsc

N = 10000
E = 320000
F = 128
NC, NS = 2, 16
NW = NC * NS
EB = 176
NG = 58
IC = 8
EROWS = NW * NG
EPAD = EROWS * EB
NAE = 10240
NBUF = 4
ZCH = 128
NZK = NAE // (ZCH * NS)

_mesh = plsc.VectorSubcoreMesh(core_axis_name="c", subcore_axis_name="s")


@functools.partial(
    pl.kernel,
    out_type=jax.ShapeDtypeStruct((NC, NAE, F), jnp.float32),
    mesh=_mesh,
    scratch_types=[
        [pltpu.VMEM((EB,), jnp.int32)] * 2,
        [pltpu.VMEM((EB,), jnp.int32)] * 2,
        [pltpu.VMEM((EB, F), jnp.float32)] * 2,
        pltpu.VMEM_SHARED((NAE, F), jnp.float32),
        [pltpu.SemaphoreType.DMA] * NBUF,
        pltpu.SemaphoreType.DMA,
    ],
)
def _sc_agg(x_hbm, src_hbm, dst_hbm, z_hbm, out_hbm,
            sidx, didx, rows, acc, sems, sem_i):
    cid = lax.axis_index("c")
    sid = lax.axis_index("s")
    wid = cid * NS + sid
    rbase = wid * NG

    for k in range(NZK):
        if k >= NBUF:
            pltpu.make_async_copy(z_hbm.at[pl.ds(0, ZCH)],
                                  acc.at[pl.ds(0, ZCH)], sems[k % NBUF]).wait()
        row = (k * NS + sid) * ZCH
        pltpu.async_copy(z_hbm.at[pl.ds(row, ZCH)],
                         acc.at[pl.ds(row, ZCH)], sems[k % NBUF])
    for b in range(NBUF):
        pltpu.make_async_copy(z_hbm.at[pl.ds(0, ZCH)],
                              acc.at[pl.ds(0, ZCH)], sems[b]).wait()
    plsc.subcore_barrier()

    ebase = wid * (NG * EB)
    pltpu.sync_copy(src_hbm.at[pl.ds(ebase, EB)], sidx[0])
    pltpu.sync_copy(dst_hbm.at[pl.ds(ebase, EB)], didx[0])

    pltpu.async_copy(x_hbm.at[sidx[0]], rows[0], sems[0])

    def _batch2(i, carry):
        for b in range(2):
            g = i * 2 + b
            nxt = 1 - b

            @pl.when(g + 1 < NG)
            def _():
                off = ebase + (g + 1) * EB
                pltpu.async_copy(src_hbm.at[pl.ds(off, EB)], sidx[nxt], sem_i)
                pltpu.async_copy(dst_hbm.at[pl.ds(off, EB)], didx[nxt], sem_i)

            pltpu.make_async_copy(x_hbm.at[sidx[b]], rows[b], sems[0]).wait()
            pltpu.async_copy(rows[b], acc.at[didx[b]], sems[1], add=True)

            @pl.when(g + 1 < NG)
            def _():
                pltpu.make_async_copy(src_hbm.at[pl.ds(0, EB)], sidx[nxt],
                                      sem_i).wait()
                pltpu.make_async_copy(dst_hbm.at[pl.ds(0, EB)], didx[nxt],
                                      sem_i).wait()
                pltpu.async_copy(x_hbm.at[sidx[nxt]], rows[nxt], sems[0])

            pltpu.make_async_copy(rows[b], acc.at[pl.ds(0, EB)], sems[1]).wait()
        return carry

    lax.fori_loop(0, NG // 2, _batch2, 0)
    plsc.subcore_barrier()

    for k in range(NZK):
        if k >= NBUF:
            pltpu.make_async_copy(acc.at[pl.ds(0, ZCH)],
                                  out_hbm.at[cid, pl.ds(0, ZCH)], sems[k % NBUF]).wait()
        row = (k * NS + sid) * ZCH
        pltpu.async_copy(acc.at[pl.ds(row, ZCH)],
                         out_hbm.at[cid, pl.ds(row, ZCH)], sems[k % NBUF])
    for b in range(NBUF):
        pltpu.make_async_copy(acc.at[pl.ds(0, ZCH)],
                              out_hbm.at[cid, pl.ds(0, ZCH)], sems[b]).wait()


def _tc_layer_call(y, x, W, Ws, b, *, final):
    M, Fin = x.shape
    Fo = W.shape[1]
    BM = 2000

    def body(ya_ref, yb_ref, x_ref, W_ref, Ws_ref, b_ref, o_ref):
        ys = ya_ref[0] + yb_ref[0]
        acc = jnp.dot(ys, W_ref[...], preferred_element_type=jnp.float32)
        acc += jnp.dot(x_ref[...], Ws_ref[...], preferred_element_type=jnp.float32)
        logits = acc + b_ref[...]
        if final:
            m = jnp.max(logits, axis=-1, keepdims=True)
            z = logits - m
            lse = jnp.log(jnp.sum(jnp.exp(z), axis=-1, keepdims=True))
            o_ref[...] = z - lse
        else:
            o_ref[...] = logits

    return pl.pallas_call(
        body,
        grid=(M // BM,),
        in_specs=[
            pl.BlockSpec((1, BM, Fin), lambda i: (0, i, 0)),
            pl.BlockSpec((1, BM, Fin), lambda i: (1, i, 0)),
            pl.BlockSpec((BM, Fin), lambda i: (i, 0)),
            pl.BlockSpec((Fin, Fo), lambda i: (0, 0)),
            pl.BlockSpec((Fin, Fo), lambda i: (0, 0)),
            pl.BlockSpec((1, Fo), lambda i: (0, 0)),
        ],
        out_specs=pl.BlockSpec((BM, Fo), lambda i: (i, 0)),
        out_shape=jax.ShapeDtypeStruct((M, Fo), jnp.float32),
    )(y, y, x, W, Ws, b.reshape(1, Fo))


def kernel(fea, edge_index, W_in, Ws_in, b_in, W_mid, Ws_mid, b_mid,
           W_out, Ws_out, b_out):
    pad = EPAD - E
    ar = jnp.arange(pad, dtype=jnp.int32)
    src = jnp.concatenate([edge_index[0], jax.lax.rem(ar * 41, N)])
    dst = jnp.concatenate([edge_index[1], N + jax.lax.rem(ar, NAE - N)])

    z = jnp.zeros((NAE, F), jnp.float32)
    y = _sc_agg(fea, src, dst, z)
    x1 = _tc_layer_call(y, fea, W_in, Ws_in, b_in, final=False)
    y = _sc_agg(x1, src, dst, z)
    x2 = _tc_layer_call(y, x1, W_mid, Ws_mid, b_mid, final=False)
    y = _sc_agg(x2, src, dst, z)
    return _tc_layer_call(y, x2, W_out, Ws_out, b_out, final=True)
